# baseline (speedup 1.0000x reference)
import dataclasses
import functools

import jax
import jax.numpy as jnp
from jax import lax
from jax.experimental import pallas as pl
from jax.experimental.pallas import tpu as pltpu
from jax.experimental.pallas import tpu_sc as plsc

N = 10000
E = 320000
D = 128
R = 4

NC = 2
NS = 16
NW = NC * NS
L = 16

CHUNK = 64
NBUF = 4
LAG = 3
TC_CHUNKS = 160
EPT = TC_CHUNKS * CHUNK
E_PAD = NW * EPT
TOTAL_CH = NW * TC_CHUNKS

K0 = 224
K1 = TC_CHUNKS * 2 - K0

NBINS = 40960
PAD_BIN = R * N
ACC_ROWS = 10240

BN = 2000

_mesh = plsc.VectorSubcoreMesh(core_axis_name="c", subcore_axis_name="s")

_sc_params = pltpu.CompilerParams()
if "needs_layout_passes" in pltpu.CompilerParams.__dataclass_fields__:
  _sc_params = dataclasses.replace(_sc_params, needs_layout_passes=False)


def _matmul5(x, W5):
  def body(x_ref, w_ref, o_ref):
    o_ref[0] = jnp.dot(x_ref[...].astype(jnp.bfloat16),
                       w_ref[0].astype(jnp.bfloat16),
                       preferred_element_type=jnp.float32)

  return pl.pallas_call(
      body,
      grid=(R + 1, N // BN),
      in_specs=[
          pl.BlockSpec((BN, D), lambda r, i: (i, 0)),
          pl.BlockSpec((1, D, D), lambda r, i: (r, 0, 0)),
      ],
      out_specs=pl.BlockSpec((1, BN, D), lambda r, i: (r, i, 0)),
      out_shape=jax.ShapeDtypeStruct((R + 1, N, D), jnp.float32),
  )(x, W5)


@functools.partial(
    pl.kernel,
    out_type=jax.ShapeDtypeStruct((NC, NBINS), jnp.float32),
    mesh=_mesh,
    scratch_types=[
        pltpu.VMEM((TC_CHUNKS, 3 * CHUNK), jnp.int32),
        pltpu.VMEM((1, CHUNK), jnp.int32),
        pltpu.VMEM((CHUNK,), jnp.float32),
        pltpu.VMEM((NBINS // NS,), jnp.float32),
        pltpu.VMEM_SHARED((NBINS,), jnp.float32),
        pltpu.SemaphoreType.DMA,
    ],
    compiler_params=_sc_params,
)
def _counts_kernel(edata, out, ev, combv, onesv, zbuf, csh, sem):
  c = lax.axis_index("c")
  s = lax.axis_index("s")
  wid = c * NS + s
  pltpu.async_copy(
      edata.at[pl.ds(wid * TC_CHUNKS, TC_CHUNKS)], ev, sem).wait()

  @pl.loop(0, CHUNK // L)
  def _(i):
    onesv[pl.ds(i * L, L)] = jnp.full((L,), 1.0, jnp.float32)

  SL = NBINS // NS

  @pl.loop(0, SL // L)
  def _(i):
    zbuf[pl.ds(i * L, L)] = jnp.zeros((L,), jnp.float32)

  pltpu.sync_copy(zbuf, csh.at[pl.ds(s * SL, SL)])
  plsc.subcore_barrier()

  @pl.loop(0, TC_CHUNKS)
  def _(t):
    @pl.loop(0, CHUNK // L)
    def _(g):
      combv[0, pl.ds(g * L, L)] = (
          ev[t, pl.ds(2 * CHUNK + g * L, L)] * R
          + ev[t, pl.ds(CHUNK + g * L, L)])

    pltpu.sync_copy(onesv, csh.at[combv.at[0]], add=True)

  plsc.subcore_barrier()
  pltpu.sync_copy(csh.at[pl.ds(s * SL, SL)], out.at[c, pl.ds(s * SL, SL)])


_NSL = NBINS // NS
_NSUB = 640


@functools.partial(
    pl.kernel,
    out_type=jax.ShapeDtypeStruct((NC, ACC_ROWS, D), jnp.float32),
    mesh=_mesh,
    scratch_types=[
        pltpu.VMEM((NBUF, 3 * CHUNK), jnp.int32),
        pltpu.VMEM((NBUF, CHUNK), jnp.int32),
        pltpu.VMEM((NBUF, CHUNK), jnp.int32),
        pltpu.VMEM((NBUF, CHUNK), jnp.int32),
        pltpu.VMEM((NBUF, CHUNK), jnp.float32),
        pltpu.VMEM((NBUF, CHUNK, D), jnp.float32),
        pltpu.VMEM((_NSUB,), jnp.float32),
        pltpu.VMEM((_NSUB,), jnp.float32),
        pltpu.VMEM((_NSUB,), jnp.float32),
        pltpu.VMEM_SHARED((NBINS,), jnp.float32),
        pltpu.VMEM_SHARED((ACC_ROWS, D), jnp.float32),
        pltpu.SemaphoreType.DMA((NBUF,)),
        pltpu.SemaphoreType.DMA((NBUF,)),
        pltpu.SemaphoreType.DMA((NBUF,)),
        pltpu.SemaphoreType.DMA((NBUF,)),
    ],
    compiler_params=_sc_params,
)
def _edge_kernel(edata, cnts, h4, out,
                 ev, idxv, combv, dstv, scalev, rows,
                 t0, t1, nbuf, normsh, acc, esem, gsem, vsem, csem):
  c = lax.axis_index("c")
  s = lax.axis_index("s")
  start = jnp.where(c == 0, s * K0, NS * K0 + s * K1)
  kc = jnp.where(c == 0, K0, K1)

  @pl.loop(0, _NSL // _NSUB)
  def _(u):
    base = s * _NSL + u * _NSUB
    pltpu.async_copy(cnts.at[0, pl.ds(base, _NSUB)], t0, esem.at[0]).wait()
    pltpu.async_copy(cnts.at[1, pl.ds(base, _NSUB)], t1, esem.at[0]).wait()

    @pl.loop(0, _NSUB // L)
    def _(i):
      cn = t0[pl.ds(i * L, L)] + t1[pl.ds(i * L, L)]
      nv = 1.0 / jnp.maximum(cn, 1.0)
      gidx = base + i * L + lax.broadcasted_iota(jnp.int32, (L,), 0)
      nbuf[pl.ds(i * L, L)] = jnp.where(gidx < PAD_BIN, nv, 0.0)

    pltpu.sync_copy(nbuf, normsh.at[pl.ds(base, _NSUB)])

  @pl.loop(0, CHUNK)
  def _(i):
    for j in range(D // L):
      rows[0, i, pl.ds(j * L, L)] = jnp.zeros((L,), jnp.float32)

  RPT = ACC_ROWS // NS

  @pl.loop(0, RPT // CHUNK)
  def _(b):
    pltpu.sync_copy(rows.at[0], acc.at[pl.ds(s * RPT + b * CHUNK, CHUNK)])

  plsc.subcore_barrier()

  def issue_edata(t, p):
    pltpu.async_copy(edata.at[start + t], ev.at[p], esem.at[p])

  def wait_edata(t, p):
    pltpu.make_async_copy(edata.at[start + t], ev.at[p], esem.at[p]).wait()

  def compute_indices(p):
    @pl.loop(0, CHUNK // L)
    def _(g):
      sl = pl.ds(g * L, L)
      tv = ev[p, pl.ds(CHUNK + g * L, L)]
      dv = ev[p, pl.ds(2 * CHUNK + g * L, L)]
      idxv[p, sl] = tv * N + ev[p, pl.ds(g * L, L)]
      combv[p, sl] = dv * R + tv
      dstv[p, sl] = dv

  def issue_gathers(p):
    pltpu.async_copy(h4.at[idxv.at[p]], rows.at[p], gsem.at[p])
    pltpu.async_copy(normsh.at[combv.at[p]], scalev.at[p], vsem.at[p])

  def wait_gathers(p):
    pltpu.make_async_copy(h4.at[idxv.at[p]], rows.at[p], gsem.at[p]).wait()
    pltpu.make_async_copy(
        normsh.at[combv.at[p]], scalev.at[p], vsem.at[p]).wait()

  def multiply(p):
    @pl.loop(0, CHUNK)
    def _(e):
      sc = plsc.load_gather(scalev.at[p], [jnp.full((L,), e, jnp.int32)])
      for j in range(D // L):
        rows[p, e, pl.ds(j * L, L)] = rows[p, e, pl.ds(j * L, L)] * sc

  def issue_scatter(p):
    pltpu.async_copy(rows.at[p], acc.at[dstv.at[p]], csem.at[p], add=True)

  def wait_scatter(p):
    pltpu.make_async_copy(rows.at[p], acc.at[dstv.at[p]], csem.at[p]).wait()

  for p in range(NBUF):
    issue_edata(p, p)

  @pl.loop(0, kc // NBUF)
  def _(k):
    for sub in range(NBUF):
      p = sub
      t = NBUF * k + sub

      wait_edata(t, p)

      @pl.when(k > 0)
      def _():
        wait_scatter(p)

      compute_indices(p)
      issue_gathers(p)

      @pl.when(k < kc // NBUF - 1)
      def _():
        issue_edata(t + NBUF, p)

      p2 = (sub + NBUF - LAG) % NBUF
      if sub >= LAG:
        wait_gathers(p2)
        multiply(p2)
        issue_scatter(p2)
      else:
        @pl.when(k > 0)
        def _():
          wait_gathers(p2)
          multiply(p2)
          issue_scatter(p2)

  for i in range(LAG):
    p2 = (NBUF - LAG + i) % NBUF
    wait_gathers(p2)
    multiply(p2)
    issue_scatter(p2)
  for p in range(NBUF):
    wait_scatter(p)

  plsc.subcore_barrier()

  @pl.loop(0, RPT // 128)
  def _(b):
    pltpu.sync_copy(acc.at[pl.ds(s * RPT + b * 128, 128)],
                    out.at[c, pl.ds(s * RPT + b * 128, 128)])


def _finish(partial_acc, hr, bias2):
  def body(a0_ref, a1_ref, hr_ref, b_ref, o_ref):
    o_ref[...] = jnp.tanh(
        a0_ref[0] + a1_ref[0] + hr_ref[...] + b_ref[...])

  return pl.pallas_call(
      body,
      grid=(N // BN,),
      in_specs=[
          pl.BlockSpec((1, BN, D), lambda i: (0, i, 0)),
          pl.BlockSpec((1, BN, D), lambda i: (1, i, 0)),
          pl.BlockSpec((BN, D), lambda i: (i, 0)),
          pl.BlockSpec((1, D), lambda i: (0, 0)),
      ],
      out_specs=pl.BlockSpec((BN, D), lambda i: (i, 0)),
      out_shape=jax.ShapeDtypeStruct((N, D), jnp.float32),
  )(partial_acc, partial_acc, hr, bias2)


def kernel(x, edge_index, edge_type, W, root, bias):
  src = edge_index[0]
  dst = edge_index[1]
  pad = E_PAD - E
  src3 = jnp.pad(src, (0, pad)).reshape(TOTAL_CH, CHUNK)
  typ3 = jnp.pad(edge_type, (0, pad)).reshape(TOTAL_CH, CHUNK)
  dst3 = jnp.pad(dst, (0, pad), constant_values=N).reshape(TOTAL_CH, CHUNK)
  edata = jnp.concatenate([src3, typ3, dst3], axis=1)

  W5 = jnp.concatenate([W, root[None]], axis=0)
  h5 = _matmul5(x, W5)
  h4 = h5[:R].reshape(R * N, D)

  counts = _counts_kernel(edata)
  partial_acc = _edge_kernel(edata, counts, h4)

  return _finish(partial_acc, h5[R], bias.reshape(1, D))


# speedup vs baseline: 1.0024x; 1.0024x over previous
import dataclasses
import functools

import jax
import jax.numpy as jnp
from jax import lax
from jax.experimental import pallas as pl
from jax.experimental.pallas import tpu as pltpu
from jax.experimental.pallas import tpu_sc as plsc

N = 10000
E = 320000
D = 128
R = 4

NC = 2
NS = 16
NW = NC * NS
L = 16

CHUNK = 64
NBUF = 4
LAG = 3
TC_CHUNKS = 160
EPT = TC_CHUNKS * CHUNK
E_PAD = NW * EPT
TOTAL_CH = NW * TC_CHUNKS

K0 = 224
K1 = TC_CHUNKS * 2 - K0

NBINS = 40960
PAD_BIN = R * N
ACC_ROWS = 10240

BN = 2000

_mesh = plsc.VectorSubcoreMesh(core_axis_name="c", subcore_axis_name="s")

_sc_params = pltpu.CompilerParams()
if "needs_layout_passes" in pltpu.CompilerParams.__dataclass_fields__:
  _sc_params = dataclasses.replace(_sc_params, needs_layout_passes=False)


def _matmul5(x, W5):
  def body(x_ref, w_ref, o_ref):
    o_ref[0] = jnp.dot(x_ref[...], w_ref[0], preferred_element_type=jnp.float32)

  return pl.pallas_call(
      body,
      grid=(R + 1, N // BN),
      in_specs=[
          pl.BlockSpec((BN, D), lambda r, i: (i, 0)),
          pl.BlockSpec((1, D, D), lambda r, i: (r, 0, 0)),
      ],
      out_specs=pl.BlockSpec((1, BN, D), lambda r, i: (r, i, 0)),
      out_shape=jax.ShapeDtypeStruct((R + 1, N, D), jnp.float32),
  )(x, W5)


@functools.partial(
    pl.kernel,
    out_type=jax.ShapeDtypeStruct((NC, NBINS), jnp.float32),
    mesh=_mesh,
    scratch_types=[
        pltpu.VMEM((TC_CHUNKS, 3 * CHUNK), jnp.int32),
        pltpu.VMEM((1, CHUNK), jnp.int32),
        pltpu.VMEM((CHUNK,), jnp.float32),
        pltpu.VMEM((NBINS // NS,), jnp.float32),
        pltpu.VMEM_SHARED((NBINS,), jnp.float32),
        pltpu.SemaphoreType.DMA,
    ],
    compiler_params=_sc_params,
)
def _counts_kernel(edata, out, ev, combv, onesv, zbuf, csh, sem):
  c = lax.axis_index("c")
  s = lax.axis_index("s")
  wid = c * NS + s
  pltpu.async_copy(
      edata.at[pl.ds(wid * TC_CHUNKS, TC_CHUNKS)], ev, sem).wait()

  @pl.loop(0, CHUNK // L)
  def _(i):
    onesv[pl.ds(i * L, L)] = jnp.full((L,), 1.0, jnp.float32)

  SL = NBINS // NS

  @pl.loop(0, SL // L)
  def _(i):
    zbuf[pl.ds(i * L, L)] = jnp.zeros((L,), jnp.float32)

  pltpu.sync_copy(zbuf, csh.at[pl.ds(s * SL, SL)])
  plsc.subcore_barrier()

  @pl.loop(0, TC_CHUNKS)
  def _(t):
    @pl.loop(0, CHUNK // L)
    def _(g):
      combv[0, pl.ds(g * L, L)] = (
          ev[t, pl.ds(2 * CHUNK + g * L, L)] * R
          + ev[t, pl.ds(CHUNK + g * L, L)])

    pltpu.sync_copy(onesv, csh.at[combv.at[0]], add=True)

  plsc.subcore_barrier()
  pltpu.sync_copy(csh.at[pl.ds(s * SL, SL)], out.at[c, pl.ds(s * SL, SL)])


_NSL = NBINS // NS
_NSUB = 640


@functools.partial(
    pl.kernel,
    out_type=jax.ShapeDtypeStruct((NC, ACC_ROWS, D), jnp.float32),
    mesh=_mesh,
    scratch_types=[
        pltpu.VMEM((NBUF, 3 * CHUNK), jnp.int32),
        pltpu.VMEM((NBUF, CHUNK), jnp.int32),
        pltpu.VMEM((NBUF, CHUNK), jnp.int32),
        pltpu.VMEM((NBUF, CHUNK), jnp.int32),
        pltpu.VMEM((NBUF, CHUNK), jnp.float32),
        pltpu.VMEM((NBUF, CHUNK, D), jnp.float32),
        pltpu.VMEM((_NSUB,), jnp.float32),
        pltpu.VMEM((_NSUB,), jnp.float32),
        pltpu.VMEM((_NSUB,), jnp.float32),
        pltpu.VMEM_SHARED((NBINS,), jnp.float32),
        pltpu.VMEM_SHARED((ACC_ROWS, D), jnp.float32),
        pltpu.SemaphoreType.DMA((NBUF,)),
        pltpu.SemaphoreType.DMA((NBUF,)),
        pltpu.SemaphoreType.DMA((NBUF,)),
        pltpu.SemaphoreType.DMA((NBUF,)),
    ],
    compiler_params=_sc_params,
)
def _edge_kernel(edata, cnts, h4, out,
                 ev, idxv, combv, dstv, scalev, rows,
                 t0, t1, nbuf, normsh, acc, esem, gsem, vsem, csem):
  c = lax.axis_index("c")
  s = lax.axis_index("s")
  start = jnp.where(c == 0, s * K0, NS * K0 + s * K1)
  kc = jnp.where(c == 0, K0, K1)

  @pl.loop(0, _NSL // _NSUB)
  def _(u):
    base = s * _NSL + u * _NSUB
    pltpu.async_copy(cnts.at[0, pl.ds(base, _NSUB)], t0, esem.at[0]).wait()
    pltpu.async_copy(cnts.at[1, pl.ds(base, _NSUB)], t1, esem.at[0]).wait()

    @pl.loop(0, _NSUB // L)
    def _(i):
      cn = t0[pl.ds(i * L, L)] + t1[pl.ds(i * L, L)]
      nv = 1.0 / jnp.maximum(cn, 1.0)
      gidx = base + i * L + lax.broadcasted_iota(jnp.int32, (L,), 0)
      nbuf[pl.ds(i * L, L)] = jnp.where(gidx < PAD_BIN, nv, 0.0)

    pltpu.sync_copy(nbuf, normsh.at[pl.ds(base, _NSUB)])

  @pl.loop(0, CHUNK)
  def _(i):
    for j in range(D // L):
      rows[0, i, pl.ds(j * L, L)] = jnp.zeros((L,), jnp.float32)

  RPT = ACC_ROWS // NS

  @pl.loop(0, RPT // CHUNK)
  def _(b):
    pltpu.sync_copy(rows.at[0], acc.at[pl.ds(s * RPT + b * CHUNK, CHUNK)])

  plsc.subcore_barrier()

  def issue_edata(t, p):
    pltpu.async_copy(edata.at[start + t], ev.at[p], esem.at[p])

  def wait_edata(t, p):
    pltpu.make_async_copy(edata.at[start + t], ev.at[p], esem.at[p]).wait()

  def compute_indices(p):
    @pl.loop(0, CHUNK // L)
    def _(g):
      sl = pl.ds(g * L, L)
      tv = ev[p, pl.ds(CHUNK + g * L, L)]
      dv = ev[p, pl.ds(2 * CHUNK + g * L, L)]
      idxv[p, sl] = tv * N + ev[p, pl.ds(g * L, L)]
      combv[p, sl] = dv * R + tv
      dstv[p, sl] = dv

  def issue_gathers(p):
    pltpu.async_copy(h4.at[idxv.at[p]], rows.at[p], gsem.at[p])
    pltpu.async_copy(normsh.at[combv.at[p]], scalev.at[p], vsem.at[p])

  def wait_gathers(p):
    pltpu.make_async_copy(h4.at[idxv.at[p]], rows.at[p], gsem.at[p]).wait()
    pltpu.make_async_copy(
        normsh.at[combv.at[p]], scalev.at[p], vsem.at[p]).wait()

  def multiply(p):
    @pl.loop(0, CHUNK)
    def _(e):
      sc = plsc.load_gather(scalev.at[p], [jnp.full((L,), e, jnp.int32)])
      for j in range(D // L):
        rows[p, e, pl.ds(j * L, L)] = rows[p, e, pl.ds(j * L, L)] * sc

  def issue_scatter(p):
    pltpu.async_copy(rows.at[p], acc.at[dstv.at[p]], csem.at[p], add=True)

  def wait_scatter(p):
    pltpu.make_async_copy(rows.at[p], acc.at[dstv.at[p]], csem.at[p]).wait()

  for p in range(NBUF):
    issue_edata(p, p)

  @pl.loop(0, kc // NBUF)
  def _(k):
    for sub in range(NBUF):
      p = sub
      t = NBUF * k + sub

      wait_edata(t, p)

      @pl.when(k > 0)
      def _():
        wait_scatter(p)

      compute_indices(p)
      issue_gathers(p)

      @pl.when(k < kc // NBUF - 1)
      def _():
        issue_edata(t + NBUF, p)

      p2 = (sub + NBUF - LAG) % NBUF
      if sub >= LAG:
        wait_gathers(p2)
        multiply(p2)
        issue_scatter(p2)
      else:
        @pl.when(k > 0)
        def _():
          wait_gathers(p2)
          multiply(p2)
          issue_scatter(p2)

  for i in range(LAG):
    p2 = (NBUF - LAG + i) % NBUF
    wait_gathers(p2)
    multiply(p2)
    issue_scatter(p2)
  for p in range(NBUF):
    wait_scatter(p)

  plsc.subcore_barrier()

  @pl.loop(0, RPT // 128)
  def _(b):
    pltpu.sync_copy(acc.at[pl.ds(s * RPT + b * 128, 128)],
                    out.at[c, pl.ds(s * RPT + b * 128, 128)])


def _finish(partial_acc, hr, bias2):
  def body(a0_ref, a1_ref, hr_ref, b_ref, o_ref):
    o_ref[...] = jnp.tanh(
        a0_ref[0] + a1_ref[0] + hr_ref[...] + b_ref[...])

  return pl.pallas_call(
      body,
      grid=(N // BN,),
      in_specs=[
          pl.BlockSpec((1, BN, D), lambda i: (0, i, 0)),
          pl.BlockSpec((1, BN, D), lambda i: (1, i, 0)),
          pl.BlockSpec((BN, D), lambda i: (i, 0)),
          pl.BlockSpec((1, D), lambda i: (0, 0)),
      ],
      out_specs=pl.BlockSpec((BN, D), lambda i: (i, 0)),
      out_shape=jax.ShapeDtypeStruct((N, D), jnp.float32),
  )(partial_acc, partial_acc, hr, bias2)


def kernel(x, edge_index, edge_type, W, root, bias):
  src = edge_index[0]
  dst = edge_index[1]
  pad = E_PAD - E
  src3 = jnp.pad(src, (0, pad)).reshape(TOTAL_CH, CHUNK)
  typ3 = jnp.pad(edge_type, (0, pad)).reshape(TOTAL_CH, CHUNK)
  dst3 = jnp.pad(dst, (0, pad), constant_values=N).reshape(TOTAL_CH, CHUNK)
  edata = jnp.concatenate([src3, typ3, dst3], axis=1)

  W5 = jnp.concatenate([W, root[None]], axis=0)
  h5 = _matmul5(x, W5)
  h4 = h5[:R].reshape(R * N, D)

  counts = _counts_kernel(edata)
  partial_acc = _edge_kernel(edata, counts, h4)

  return _finish(partial_acc, h5[R], bias.reshape(1, D))
